# Initial kernel scaffold; baseline (speedup 1.0000x reference)
import jax
import jax.numpy as jnp
from jax.experimental import pallas as pl


def kernel(x, edge, neighbors, W_global, W_local, W_self, bias):
    raise NotImplementedError("write your pallas kernel here")


# speedup vs baseline: 1.8482x; 1.8482x over previous
import functools

import jax
import jax.numpy as jnp
import numpy as np
from jax import lax
from jax.experimental import pallas as pl
from jax.experimental.pallas import tpu as pltpu
from jax.experimental.pallas import tpu_sc as plsc

N = 10000
DEG = 32
D = 128

NW = 32
CPW = 320
NPAD = NW * CPW
ROWS = 128
NODES_PER_STEP = ROWS // DEG
STEPS = (CPW * DEG) // ROWS


_sc_mesh = plsc.VectorSubcoreMesh(
    core_axis_name="c", subcore_axis_name="s", num_cores=2, num_subcores=16
)


@functools.partial(
    pl.kernel,
    out_type=jax.ShapeDtypeStruct((NPAD, D), jnp.float32),
    mesh=_sc_mesh,
    scratch_types=[
        pltpu.VMEM((STEPS, ROWS), jnp.int32),
        pltpu.VMEM((STEPS, ROWS), jnp.int32),
        pltpu.VMEM_SHARED((16 * CPW, D), jnp.float32),
        pltpu.VMEM((ROWS, D), jnp.float32),
        pltpu.VMEM((ROWS, D), jnp.float32),
        pltpu.SemaphoreType.DMA,
        pltpu.SemaphoreType.DMA,
    ],
)
def _sc_neighbor_sum(x_hbm, nbr_hbm, dst_hbm, zero_hbm, out_hbm,
                     nbr_v, dst_v, acc_sh, buf0, buf1, sem0, sem1):
    sid = lax.axis_index("s")
    wid = sid * 2 + lax.axis_index("c")
    pltpu.sync_copy(nbr_hbm.at[wid], nbr_v)
    pltpu.sync_copy(dst_hbm.at[sid], dst_v)
    pltpu.sync_copy(zero_hbm, acc_sh.at[pl.ds(sid * CPW, CPW)])

    pltpu.async_copy(x_hbm.at[nbr_v.at[0]], buf0, sem0)

    def step_pair(it, carry):
        a = 2 * it
        pltpu.async_copy(x_hbm.at[nbr_v.at[a + 1]], buf1, sem1)
        pltpu.make_async_copy(x_hbm.at[nbr_v.at[a]], buf0, sem0).wait()
        pltpu.sync_copy(buf0, acc_sh.at[dst_v.at[a]], add=True)
        pltpu.async_copy(x_hbm.at[nbr_v.at[a + 2]], buf0, sem0)
        pltpu.make_async_copy(x_hbm.at[nbr_v.at[a + 1]], buf1, sem1).wait()
        pltpu.sync_copy(buf1, acc_sh.at[dst_v.at[a + 1]], add=True)
        return carry

    lax.fori_loop(0, STEPS // 2 - 1, step_pair, 0)

    pltpu.async_copy(x_hbm.at[nbr_v.at[STEPS - 1]], buf1, sem1)
    pltpu.make_async_copy(x_hbm.at[nbr_v.at[STEPS - 2]], buf0, sem0).wait()
    pltpu.sync_copy(buf0, acc_sh.at[dst_v.at[STEPS - 2]], add=True)
    pltpu.make_async_copy(x_hbm.at[nbr_v.at[STEPS - 1]], buf1, sem1).wait()
    pltpu.sync_copy(buf1, acc_sh.at[dst_v.at[STEPS - 1]], add=True)

    pltpu.sync_copy(acc_sh.at[pl.ds(sid * CPW, CPW)], out_hbm.at[pl.ds(wid * CPW, CPW)])


def _tc_body(x_ref, ns_ref, wg_ref, wl_ref, ws_ref, b_ref, o_ref):
    w_sum = wg_ref[...] + ws_ref[...]
    z = lax.dot_general(x_ref[...], w_sum, (((1,), (1,)), ((), ())),
                        preferred_element_type=jnp.float32)
    z += lax.dot_general(ns_ref[...] * (1.0 / DEG), wl_ref[...],
                         (((1,), (1,)), ((), ())),
                         preferred_element_type=jnp.float32)
    z += b_ref[...]
    o_ref[...] = jnp.where(z > 0, z, jnp.exp(z) - 1.0)


_BLK = 1000

_tc_fuse = pl.pallas_call(
    _tc_body,
    grid=(N // _BLK,),
    in_specs=[
        pl.BlockSpec((_BLK, D), lambda i: (i, 0)),
        pl.BlockSpec((_BLK, D), lambda i: (i, 0)),
        pl.BlockSpec((D, D), lambda i: (0, 0)),
        pl.BlockSpec((D, D), lambda i: (0, 0)),
        pl.BlockSpec((D, D), lambda i: (0, 0)),
        pl.BlockSpec((1, D), lambda i: (0, 0)),
    ],
    out_specs=pl.BlockSpec((_BLK, D), lambda i: (i, 0)),
    out_shape=jax.ShapeDtypeStruct((N, D), jnp.float32),
)

_DST = (
    np.arange(16, dtype=np.int32)[:, None, None] * CPW
    + np.repeat(np.arange(STEPS * NODES_PER_STEP, dtype=np.int32), DEG)
      .reshape(1, STEPS, ROWS)
)


def kernel(x, edge, neighbors, W_global, W_local, W_self, bias):
    nbr = jnp.concatenate(
        [neighbors, jnp.zeros((NPAD - N) * DEG, dtype=jnp.int32)]
    ).reshape(NW, STEPS, ROWS)
    dst = jnp.asarray(_DST)
    zero = jnp.zeros((CPW, D), dtype=jnp.float32)
    ns = _sc_neighbor_sum(x, nbr, dst, zero)
    return _tc_fuse(x, ns, W_global, W_local, W_self, bias.reshape(1, D))


# speedup vs baseline: 1.8690x; 1.0113x over previous
import functools

import jax
import jax.numpy as jnp
import numpy as np
from jax import lax
from jax.experimental import pallas as pl
from jax.experimental.pallas import tpu as pltpu
from jax.experimental.pallas import tpu_sc as plsc

N = 10000
DEG = 32
D = 128

NW = 32
CPW = 320
NPAD = NW * CPW
ROWS = 128
NODES_PER_STEP = ROWS // DEG
STEPS = (CPW * DEG) // ROWS


_sc_mesh = plsc.VectorSubcoreMesh(
    core_axis_name="c", subcore_axis_name="s", num_cores=2, num_subcores=16
)


@functools.partial(
    pl.kernel,
    out_type=jax.ShapeDtypeStruct((NPAD, D), jnp.float32),
    mesh=_sc_mesh,
    scratch_types=[
        pltpu.VMEM((STEPS, ROWS), jnp.int32),
        pltpu.VMEM((STEPS, ROWS), jnp.int32),
        pltpu.VMEM_SHARED((16 * CPW, D), jnp.float32),
        pltpu.VMEM((4, ROWS, D), jnp.float32),
        [pltpu.SemaphoreType.DMA] * 4,
        [pltpu.SemaphoreType.DMA] * 4,
    ],
)
def _sc_neighbor_sum(x_hbm, nbr_hbm, dst_hbm, zero_hbm, out_hbm,
                     nbr_v, dst_v, acc_sh, bufs, gsem, ssem):
    sid = lax.axis_index("s")
    wid = sid * 2 + lax.axis_index("c")
    pltpu.sync_copy(nbr_hbm.at[wid], nbr_v)
    pltpu.sync_copy(dst_hbm.at[sid], dst_v)
    pltpu.sync_copy(zero_hbm, acc_sh.at[pl.ds(sid * CPW, CPW)])

    def g_start(j, b):
        pltpu.async_copy(x_hbm.at[nbr_v.at[j]], bufs.at[b], gsem[b])

    def g_wait(j, b):
        pltpu.make_async_copy(x_hbm.at[nbr_v.at[j]], bufs.at[b], gsem[b]).wait()

    def s_start(j, b):
        pltpu.async_copy(bufs.at[b], acc_sh.at[dst_v.at[j]], ssem[b], add=True)

    def s_wait(j, b):
        pltpu.make_async_copy(bufs.at[b], acc_sh.at[dst_v.at[j]], ssem[b]).wait()

    g_start(0, 0)
    g_start(1, 1)
    g_wait(0, 0)
    s_start(0, 0)
    g_start(2, 2)
    g_wait(1, 1)
    s_start(1, 1)
    g_start(3, 3)

    def quad(it, carry):
        j0 = 4 * it + 2
        for u in range(4):
            j = j0 + u
            b = (2 + u) % 4
            s_wait(j - 2, u)
            g_start(j + 2, u)
            g_wait(j, b)
            s_start(j, b)
        return carry

    lax.fori_loop(0, (STEPS - 4) // 4, quad, 0)

    s_wait(76, 0)
    g_wait(STEPS - 2, 2)
    s_start(STEPS - 2, 2)
    s_wait(77, 1)
    g_wait(STEPS - 1, 3)
    s_start(STEPS - 1, 3)
    s_wait(STEPS - 2, 2)
    s_wait(STEPS - 1, 3)

    pltpu.sync_copy(acc_sh.at[pl.ds(sid * CPW, CPW)], out_hbm.at[pl.ds(wid * CPW, CPW)])


def _tc_body(x_ref, ns_ref, wg_ref, wl_ref, ws_ref, b_ref, o_ref):
    w_sum = wg_ref[...] + ws_ref[...]
    z = lax.dot_general(x_ref[...], w_sum, (((1,), (1,)), ((), ())),
                        preferred_element_type=jnp.float32)
    z += lax.dot_general(ns_ref[...] * (1.0 / DEG), wl_ref[...],
                         (((1,), (1,)), ((), ())),
                         preferred_element_type=jnp.float32)
    z += b_ref[...]
    o_ref[...] = jnp.where(z > 0, z, jnp.exp(z) - 1.0)


_BLK = 1000

_tc_fuse = pl.pallas_call(
    _tc_body,
    grid=(N // _BLK,),
    in_specs=[
        pl.BlockSpec((_BLK, D), lambda i: (i, 0)),
        pl.BlockSpec((_BLK, D), lambda i: (i, 0)),
        pl.BlockSpec((D, D), lambda i: (0, 0)),
        pl.BlockSpec((D, D), lambda i: (0, 0)),
        pl.BlockSpec((D, D), lambda i: (0, 0)),
        pl.BlockSpec((1, D), lambda i: (0, 0)),
    ],
    out_specs=pl.BlockSpec((_BLK, D), lambda i: (i, 0)),
    out_shape=jax.ShapeDtypeStruct((N, D), jnp.float32),
)

_DST = (
    np.arange(16, dtype=np.int32)[:, None, None] * CPW
    + np.repeat(np.arange(STEPS * NODES_PER_STEP, dtype=np.int32), DEG)
      .reshape(1, STEPS, ROWS)
)


def kernel(x, edge, neighbors, W_global, W_local, W_self, bias):
    nbr = jnp.concatenate(
        [neighbors, jnp.zeros((NPAD - N) * DEG, dtype=jnp.int32)]
    ).reshape(NW, STEPS, ROWS)
    dst = jnp.asarray(_DST)
    zero = jnp.zeros((CPW, D), dtype=jnp.float32)
    ns = _sc_neighbor_sum(x, nbr, dst, zero)
    return _tc_fuse(x, ns, W_global, W_local, W_self, bias.reshape(1, D))


# speedup vs baseline: 3.8565x; 2.0634x over previous
import functools

import jax
import jax.numpy as jnp
import numpy as np
from jax import lax
from jax.experimental import pallas as pl
from jax.experimental.pallas import tpu as pltpu
from jax.experimental.pallas import tpu_sc as plsc

N = 10000
DEG = 32
D = 128

NW = 32
CPW = 320
NPAD = NW * CPW
ROWS = 64
NODES_PER_STEP = ROWS // DEG
STEPS = (CPW * DEG) // ROWS
PHASES = 4
SPP = STEPS // PHASES
HPW = CPW // PHASES
XSH = 10112


_sc_mesh = plsc.VectorSubcoreMesh(
    core_axis_name="c", subcore_axis_name="s", num_cores=2, num_subcores=16
)


@functools.partial(
    pl.kernel,
    out_type=jax.ShapeDtypeStruct((NPAD, D), jnp.float32),
    mesh=_sc_mesh,
    scratch_types=[
        pltpu.VMEM((SPP, ROWS), jnp.int32),
        pltpu.VMEM((SPP, ROWS), jnp.int32),
        pltpu.VMEM_SHARED((16 * HPW, D), jnp.float32),
        pltpu.VMEM_SHARED((XSH, D), jnp.float32),
        pltpu.VMEM((ROWS, D), jnp.float32),
        pltpu.VMEM((ROWS, D), jnp.float32),
        pltpu.SemaphoreType.DMA,
        pltpu.SemaphoreType.DMA,
    ],
)
def _sc_neighbor_sum(x_hbm, nbr_hbm, dst_hbm, zero_hbm, out_hbm,
                     nbr_v, dst_v, acc_sh, x_sh, buf0, buf1, sem0, sem1):
    sid = lax.axis_index("s")
    wid = sid * 2 + lax.axis_index("c")
    pltpu.sync_copy(x_hbm.at[pl.ds(sid * 632, 632)], x_sh.at[pl.ds(sid * 632, 632)])
    pltpu.sync_copy(zero_hbm, acc_sh.at[pl.ds(sid * HPW, HPW)])
    plsc.subcore_barrier()

    def g_start(j, buf, sem):
        pltpu.async_copy(x_sh.at[nbr_v.at[j]], buf, sem)

    def g_wait(j, buf, sem):
        pltpu.make_async_copy(x_sh.at[nbr_v.at[j]], buf, sem).wait()

    def s_add(j, buf):
        pltpu.sync_copy(buf, acc_sh.at[dst_v.at[j]], add=True)

    def run_phase(ph):
        pltpu.sync_copy(nbr_hbm.at[wid, ph], nbr_v)
        pltpu.sync_copy(dst_hbm.at[sid, ph], dst_v)
        g_start(0, buf0, sem0)

        def pair(it, carry):
            a = 2 * it
            g_start(a + 1, buf1, sem1)
            g_wait(a, buf0, sem0)
            s_add(a, buf0)
            g_start(a + 2, buf0, sem0)
            g_wait(a + 1, buf1, sem1)
            s_add(a + 1, buf1)
            return carry

        lax.fori_loop(0, SPP // 2 - 1, pair, 0)

        g_start(SPP - 1, buf1, sem1)
        g_wait(SPP - 2, buf0, sem0)
        s_add(SPP - 2, buf0)
        g_wait(SPP - 1, buf1, sem1)
        s_add(SPP - 1, buf1)

        pltpu.sync_copy(acc_sh.at[pl.ds(sid * HPW, HPW)],
                        out_hbm.at[pl.ds(wid * CPW + ph * HPW, HPW)])

    run_phase(0)
    for _ph in range(1, PHASES):
        pltpu.sync_copy(zero_hbm, acc_sh.at[pl.ds(sid * HPW, HPW)])
        run_phase(_ph)


def _tc_body(x_ref, ns_ref, wg_ref, wl_ref, ws_ref, b_ref, o_ref):
    w_sum = wg_ref[...] + ws_ref[...]
    z = lax.dot_general(x_ref[...], w_sum, (((1,), (1,)), ((), ())),
                        preferred_element_type=jnp.float32)
    z += lax.dot_general(ns_ref[...] * (1.0 / DEG), wl_ref[...],
                         (((1,), (1,)), ((), ())),
                         preferred_element_type=jnp.float32)
    z += b_ref[...]
    o_ref[...] = jnp.where(z > 0, z, jnp.exp(z) - 1.0)


_BLK = 1000

_tc_fuse = pl.pallas_call(
    _tc_body,
    grid=(N // _BLK,),
    in_specs=[
        pl.BlockSpec((_BLK, D), lambda i: (i, 0)),
        pl.BlockSpec((_BLK, D), lambda i: (i, 0)),
        pl.BlockSpec((D, D), lambda i: (0, 0)),
        pl.BlockSpec((D, D), lambda i: (0, 0)),
        pl.BlockSpec((D, D), lambda i: (0, 0)),
        pl.BlockSpec((1, D), lambda i: (0, 0)),
    ],
    out_specs=pl.BlockSpec((_BLK, D), lambda i: (i, 0)),
    out_shape=jax.ShapeDtypeStruct((N, D), jnp.float32),
)

_DST = (
    np.arange(16, dtype=np.int32)[:, None, None, None] * HPW
    + np.repeat(np.arange(SPP * NODES_PER_STEP, dtype=np.int32), DEG)
      .reshape(1, 1, SPP, ROWS)
    + np.zeros((1, PHASES, 1, 1), dtype=np.int32)
)


def kernel(x, edge, neighbors, W_global, W_local, W_self, bias):
    x_pad = jnp.concatenate([x, jnp.zeros((XSH - N, D), dtype=jnp.float32)])
    nbr = jnp.concatenate(
        [neighbors, jnp.zeros((NPAD - N) * DEG, dtype=jnp.int32)]
    ).reshape(NW, PHASES, SPP, ROWS)
    dst = jnp.asarray(_DST)
    zero = jnp.zeros((HPW, D), dtype=jnp.float32)
    ns = _sc_neighbor_sum(x_pad, nbr, dst, zero)
    return _tc_fuse(x, ns, W_global, W_local, W_self, bias.reshape(1, D))


# speedup vs baseline: 6.2925x; 1.6317x over previous
import functools

import jax
import jax.numpy as jnp
from jax import lax
from jax.experimental import pallas as pl
from jax.experimental.pallas import tpu as pltpu
from jax.experimental.pallas import tpu_sc as plsc

N = 10000
DEG = 32
D = 128
LANES = 16
VECS = D // LANES

NW = 32
CPW = 320
NPAD = NW * CPW
ROWS = 64
NODES_PER_STEP = ROWS // DEG
STEPS = (CPW * DEG) // ROWS
PHASES = 4
SPP = STEPS // PHASES
HPW = CPW // PHASES
XSH = 10112


_sc_mesh = plsc.VectorSubcoreMesh(
    core_axis_name="c", subcore_axis_name="s", num_cores=2, num_subcores=16
)


@functools.partial(
    pl.kernel,
    out_type=jax.ShapeDtypeStruct((NPAD, D), jnp.float32),
    mesh=_sc_mesh,
    scratch_types=[
        pltpu.VMEM((SPP, ROWS), jnp.int32),
        pltpu.VMEM((HPW, D), jnp.float32),
        pltpu.VMEM_SHARED((XSH, D), jnp.float32),
        pltpu.VMEM((ROWS, D), jnp.float32),
        pltpu.VMEM((ROWS, D), jnp.float32),
        pltpu.SemaphoreType.DMA,
        pltpu.SemaphoreType.DMA,
        pltpu.SemaphoreType.DMA,
    ],
)
def _sc_neighbor_sum(x_hbm, nbr_hbm, out_hbm,
                     nbr_v, res_v, x_sh, buf0, buf1, gsem0, gsem1, wsem):
    sid = lax.axis_index("s")
    wid = sid * 2 + lax.axis_index("c")
    pltpu.sync_copy(x_hbm.at[pl.ds(sid * 632, 632)], x_sh.at[pl.ds(sid * 632, 632)])
    plsc.subcore_barrier()

    def g_start(j, buf, sem):
        pltpu.async_copy(x_sh.at[nbr_v.at[j]], buf, sem)

    def g_wait(j, buf, sem):
        pltpu.make_async_copy(x_sh.at[nbr_v.at[j]], buf, sem).wait()

    def accumulate(j, buf):
        for n in range(NODES_PER_STEP):
            def grp(g, carry):
                base = n * DEG + g * 8
                new = []
                for c in range(VECS):
                    sl = pl.ds(c * LANES, LANES)
                    v = [buf[base + k, sl] for k in range(8)]
                    gs = ((v[0] + v[1]) + (v[2] + v[3])) + \
                         ((v[4] + v[5]) + (v[6] + v[7]))
                    new.append(carry[c] + gs)
                return tuple(new)

            tot = lax.fori_loop(
                0, DEG // 8, grp,
                tuple(jnp.zeros((LANES,), jnp.float32) for _ in range(VECS)))
            for c in range(VECS):
                res_v[NODES_PER_STEP * j + n, pl.ds(c * LANES, LANES)] = tot[c]

    def run_phase(ph):
        pltpu.sync_copy(nbr_hbm.at[wid, ph], nbr_v)
        g_start(0, buf0, gsem0)

        def pair(it, carry):
            a = 2 * it
            g_start(a + 1, buf1, gsem1)
            g_wait(a, buf0, gsem0)
            accumulate(a, buf0)
            g_start(a + 2, buf0, gsem0)
            g_wait(a + 1, buf1, gsem1)
            accumulate(a + 1, buf1)
            return carry

        lax.fori_loop(0, SPP // 2 - 1, pair, 0)

        g_start(SPP - 1, buf1, gsem1)
        g_wait(SPP - 2, buf0, gsem0)
        accumulate(SPP - 2, buf0)
        g_wait(SPP - 1, buf1, gsem1)
        accumulate(SPP - 1, buf1)

        pltpu.async_copy(res_v,
                         out_hbm.at[pl.ds(wid * CPW + ph * HPW, HPW)], wsem)

    run_phase(0)
    for _ph in range(1, PHASES):
        pltpu.make_async_copy(
            res_v, out_hbm.at[pl.ds(wid * CPW, HPW)], wsem).wait()
        run_phase(_ph)
    pltpu.make_async_copy(
        res_v, out_hbm.at[pl.ds(wid * CPW, HPW)], wsem).wait()


def _tc_body(x_ref, ns_ref, wg_ref, wl_ref, ws_ref, b_ref, o_ref):
    w_sum = wg_ref[...] + ws_ref[...]
    z = lax.dot_general(x_ref[...], w_sum, (((1,), (1,)), ((), ())),
                        preferred_element_type=jnp.float32)
    z += lax.dot_general(ns_ref[...] * (1.0 / DEG), wl_ref[...],
                         (((1,), (1,)), ((), ())),
                         preferred_element_type=jnp.float32)
    z += b_ref[...]
    o_ref[...] = jnp.where(z > 0, z, jnp.exp(z) - 1.0)


_BLK = 1000

_tc_fuse = pl.pallas_call(
    _tc_body,
    grid=(N // _BLK,),
    in_specs=[
        pl.BlockSpec((_BLK, D), lambda i: (i, 0)),
        pl.BlockSpec((_BLK, D), lambda i: (i, 0)),
        pl.BlockSpec((D, D), lambda i: (0, 0)),
        pl.BlockSpec((D, D), lambda i: (0, 0)),
        pl.BlockSpec((D, D), lambda i: (0, 0)),
        pl.BlockSpec((1, D), lambda i: (0, 0)),
    ],
    out_specs=pl.BlockSpec((_BLK, D), lambda i: (i, 0)),
    out_shape=jax.ShapeDtypeStruct((N, D), jnp.float32),
)


def kernel(x, edge, neighbors, W_global, W_local, W_self, bias):
    x_pad = jnp.concatenate([x, jnp.zeros((XSH - N, D), dtype=jnp.float32)])
    nbr = jnp.concatenate(
        [neighbors, jnp.zeros((NPAD - N) * DEG, dtype=jnp.int32)]
    ).reshape(NW, PHASES, SPP, ROWS)
    ns = _sc_neighbor_sum(x_pad, nbr)
    return _tc_fuse(x, ns, W_global, W_local, W_self, bias.reshape(1, D))


# speedup vs baseline: 6.3614x; 1.0109x over previous
import functools

import jax
import jax.numpy as jnp
from jax import lax
from jax.experimental import pallas as pl
from jax.experimental.pallas import tpu as pltpu
from jax.experimental.pallas import tpu_sc as plsc

N = 10000
DEG = 32
D = 128
LANES = 16
VECS = D // LANES

NW = 32
CPW = 320
NPAD = NW * CPW
ROWS = 64
NODES_PER_STEP = ROWS // DEG
STEPS = (CPW * DEG) // ROWS
PHASES = 4
SPP = STEPS // PHASES
HPW = CPW // PHASES
XSH = 10112


_sc_mesh = plsc.VectorSubcoreMesh(
    core_axis_name="c", subcore_axis_name="s", num_cores=2, num_subcores=16
)


@functools.partial(
    pl.kernel,
    out_type=jax.ShapeDtypeStruct((NPAD, D), jnp.float32),
    mesh=_sc_mesh,
    scratch_types=[
        pltpu.VMEM((SPP, ROWS), jnp.int32),
        pltpu.VMEM((HPW, D), jnp.float32),
        pltpu.VMEM_SHARED((XSH, D), jnp.float32),
        pltpu.VMEM((ROWS, D), jnp.float32),
        pltpu.VMEM((ROWS, D), jnp.float32),
        pltpu.SemaphoreType.DMA,
        pltpu.SemaphoreType.DMA,
        pltpu.SemaphoreType.DMA,
    ],
)
def _sc_neighbor_sum(x_hbm, nbr_hbm, out_hbm,
                     nbr_v, res_v, x_sh, buf0, buf1, gsem0, gsem1, wsem):
    sid = lax.axis_index("s")
    wid = sid * 2 + lax.axis_index("c")
    @pl.when(sid < 15)
    def _():
        pltpu.sync_copy(x_hbm.at[pl.ds(sid * 632, 632)],
                        x_sh.at[pl.ds(sid * 632, 632)])

    @pl.when(sid == 15)
    def _():
        pltpu.sync_copy(x_hbm.at[pl.ds(15 * 632, N - 15 * 632)],
                        x_sh.at[pl.ds(15 * 632, N - 15 * 632)])

    plsc.subcore_barrier()

    def g_start(j, buf, sem):
        pltpu.async_copy(x_sh.at[nbr_v.at[j]], buf, sem)

    def g_wait(j, buf, sem):
        pltpu.make_async_copy(x_sh.at[nbr_v.at[j]], buf, sem).wait()

    def accumulate(j, buf):
        for n in range(NODES_PER_STEP):
            def grp(g, carry):
                base = n * DEG + g * 8
                new = []
                for c in range(VECS):
                    sl = pl.ds(c * LANES, LANES)
                    v = [buf[base + k, sl] for k in range(8)]
                    gs = ((v[0] + v[1]) + (v[2] + v[3])) + \
                         ((v[4] + v[5]) + (v[6] + v[7]))
                    new.append(carry[c] + gs)
                return tuple(new)

            tot = lax.fori_loop(
                0, DEG // 8, grp,
                tuple(jnp.zeros((LANES,), jnp.float32) for _ in range(VECS)))
            for c in range(VECS):
                res_v[NODES_PER_STEP * j + n, pl.ds(c * LANES, LANES)] = tot[c]

    def run_phase(ph):
        pltpu.sync_copy(nbr_hbm.at[wid, ph], nbr_v)
        g_start(0, buf0, gsem0)

        def pair(it, carry):
            a = 2 * it
            g_start(a + 1, buf1, gsem1)
            g_wait(a, buf0, gsem0)
            accumulate(a, buf0)
            g_start(a + 2, buf0, gsem0)
            g_wait(a + 1, buf1, gsem1)
            accumulate(a + 1, buf1)
            return carry

        lax.fori_loop(0, SPP // 2 - 1, pair, 0)

        g_start(SPP - 1, buf1, gsem1)
        g_wait(SPP - 2, buf0, gsem0)
        accumulate(SPP - 2, buf0)
        g_wait(SPP - 1, buf1, gsem1)
        accumulate(SPP - 1, buf1)

        pltpu.async_copy(res_v,
                         out_hbm.at[pl.ds(wid * CPW + ph * HPW, HPW)], wsem)

    run_phase(0)
    for _ph in range(1, PHASES):
        pltpu.make_async_copy(
            res_v, out_hbm.at[pl.ds(wid * CPW, HPW)], wsem).wait()
        run_phase(_ph)
    pltpu.make_async_copy(
        res_v, out_hbm.at[pl.ds(wid * CPW, HPW)], wsem).wait()


def _tc_self_body(x_ref, wg_ref, ws_ref, b_ref, o_ref):
    w_sum = wg_ref[...] + ws_ref[...]
    o_ref[...] = lax.dot_general(x_ref[...], w_sum, (((1,), (1,)), ((), ())),
                                 preferred_element_type=jnp.float32) + b_ref[...]


def _tc_out_body(z_ref, ns_ref, wl_ref, o_ref):
    z = z_ref[...] + lax.dot_general(ns_ref[...] * (1.0 / DEG), wl_ref[...],
                                     (((1,), (1,)), ((), ())),
                                     preferred_element_type=jnp.float32)
    o_ref[...] = jnp.where(z > 0, z, jnp.exp(z) - 1.0)


_BLK = 1000

_tc_self = pl.pallas_call(
    _tc_self_body,
    grid=(N // _BLK,),
    in_specs=[
        pl.BlockSpec((_BLK, D), lambda i: (i, 0)),
        pl.BlockSpec((D, D), lambda i: (0, 0)),
        pl.BlockSpec((D, D), lambda i: (0, 0)),
        pl.BlockSpec((1, D), lambda i: (0, 0)),
    ],
    out_specs=pl.BlockSpec((_BLK, D), lambda i: (i, 0)),
    out_shape=jax.ShapeDtypeStruct((N, D), jnp.float32),
)

_tc_out = pl.pallas_call(
    _tc_out_body,
    grid=(N // _BLK,),
    in_specs=[
        pl.BlockSpec((_BLK, D), lambda i: (i, 0)),
        pl.BlockSpec((_BLK, D), lambda i: (i, 0)),
        pl.BlockSpec((D, D), lambda i: (0, 0)),
    ],
    out_specs=pl.BlockSpec((_BLK, D), lambda i: (i, 0)),
    out_shape=jax.ShapeDtypeStruct((N, D), jnp.float32),
)


def kernel(x, edge, neighbors, W_global, W_local, W_self, bias):
    nbr = jnp.concatenate(
        [neighbors, jnp.zeros((NPAD - N) * DEG, dtype=jnp.int32)]
    ).reshape(NW, PHASES, SPP, ROWS)
    ns = _sc_neighbor_sum(x, nbr)
    z_self = _tc_self(x, W_global, W_self, bias.reshape(1, D))
    return _tc_out(z_self, ns, W_local)


# speedup vs baseline: 6.3704x; 1.0014x over previous
import functools

import jax
import jax.numpy as jnp
from jax import lax
from jax.experimental import pallas as pl
from jax.experimental.pallas import tpu as pltpu
from jax.experimental.pallas import tpu_sc as plsc

N = 10000
DEG = 32
D = 128
LANES = 16
VECS = D // LANES

NW = 32
CPW = 320
NPAD = NW * CPW
ROWS = 64
NODES_PER_STEP = ROWS // DEG
STEPS = (CPW * DEG) // ROWS
PHASES = 4
SPP = STEPS // PHASES
HPW = CPW // PHASES
XSH = 10112


_sc_mesh = plsc.VectorSubcoreMesh(
    core_axis_name="c", subcore_axis_name="s", num_cores=2, num_subcores=16
)


@functools.partial(
    pl.kernel,
    out_type=jax.ShapeDtypeStruct((NPAD, D), jnp.float32),
    mesh=_sc_mesh,
    scratch_types=[
        pltpu.VMEM((SPP, ROWS), jnp.int32),
        pltpu.VMEM((HPW, D), jnp.float32),
        pltpu.VMEM_SHARED((XSH, D), jnp.float32),
        pltpu.VMEM((ROWS, D), jnp.float32),
        pltpu.VMEM((ROWS, D), jnp.float32),
        pltpu.SemaphoreType.DMA,
        pltpu.SemaphoreType.DMA,
        pltpu.SemaphoreType.DMA,
    ],
)
def _sc_neighbor_sum(x_hbm, nbr_hbm, out_hbm,
                     nbr_v, res_v, x_sh, buf0, buf1, gsem0, gsem1, wsem):
    sid = lax.axis_index("s")
    wid = sid * 2 + lax.axis_index("c")
    @pl.when(sid < 15)
    def _():
        pltpu.sync_copy(x_hbm.at[pl.ds(sid * 632, 632)],
                        x_sh.at[pl.ds(sid * 632, 632)])

    @pl.when(sid == 15)
    def _():
        pltpu.sync_copy(x_hbm.at[pl.ds(15 * 632, N - 15 * 632)],
                        x_sh.at[pl.ds(15 * 632, N - 15 * 632)])

    plsc.subcore_barrier()

    def g_start(j, buf, sem):
        pltpu.async_copy(x_sh.at[nbr_v.at[j]], buf, sem)

    def g_wait(j, buf, sem):
        pltpu.make_async_copy(x_sh.at[nbr_v.at[j]], buf, sem).wait()

    def accumulate(j, buf):
        for n in range(NODES_PER_STEP):
            def grp(g, carry):
                base = n * DEG + g * 8
                new = []
                for c in range(VECS):
                    sl = pl.ds(c * LANES, LANES)
                    v = [buf[base + k, sl] for k in range(8)]
                    gs = ((v[0] + v[1]) + (v[2] + v[3])) + \
                         ((v[4] + v[5]) + (v[6] + v[7]))
                    new.append(carry[c] + gs)
                return tuple(new)

            tot = lax.fori_loop(
                0, DEG // 8, grp,
                tuple(jnp.zeros((LANES,), jnp.float32) for _ in range(VECS)))
            for c in range(VECS):
                res_v[NODES_PER_STEP * j + n, pl.ds(c * LANES, LANES)] = tot[c]

    def run_phase(ph):
        pltpu.sync_copy(nbr_hbm.at[wid, ph], nbr_v)
        g_start(0, buf0, gsem0)

        def pair(it, carry):
            a = 2 * it
            g_start(a + 1, buf1, gsem1)
            g_wait(a, buf0, gsem0)
            accumulate(a, buf0)
            g_start(a + 2, buf0, gsem0)
            g_wait(a + 1, buf1, gsem1)
            accumulate(a + 1, buf1)
            return carry

        lax.fori_loop(0, SPP // 2 - 1, pair, 0)

        g_start(SPP - 1, buf1, gsem1)
        g_wait(SPP - 2, buf0, gsem0)
        accumulate(SPP - 2, buf0)
        g_wait(SPP - 1, buf1, gsem1)
        accumulate(SPP - 1, buf1)

        pltpu.async_copy(res_v,
                         out_hbm.at[pl.ds(wid * CPW + ph * HPW, HPW)], wsem)

    run_phase(0)
    for _ph in range(1, PHASES):
        pltpu.make_async_copy(
            res_v, out_hbm.at[pl.ds(wid * CPW, HPW)], wsem).wait()
        run_phase(_ph)
    pltpu.make_async_copy(
        res_v, out_hbm.at[pl.ds(wid * CPW, HPW)], wsem).wait()


def _tc_self_body(x_ref, wg_ref, ws_ref, b_ref, o_ref):
    w_sum = wg_ref[...] + ws_ref[...]
    o_ref[...] = lax.dot_general(x_ref[...], w_sum, (((1,), (1,)), ((), ())),
                                 preferred_element_type=jnp.float32) + b_ref[...]


def _tc_out_body(z_ref, ns_ref, wl_ref, o_ref):
    z = z_ref[...] + lax.dot_general(ns_ref[...] * (1.0 / DEG), wl_ref[...],
                                     (((1,), (1,)), ((), ())),
                                     preferred_element_type=jnp.float32)
    o_ref[...] = jnp.where(z > 0, z, jnp.exp(z) - 1.0)


_BLK = 1000

_tc_self = pl.pallas_call(
    _tc_self_body,
    grid=(N // _BLK,),
    in_specs=[
        pl.BlockSpec((_BLK, D), lambda i: (i, 0)),
        pl.BlockSpec((D, D), lambda i: (0, 0)),
        pl.BlockSpec((D, D), lambda i: (0, 0)),
        pl.BlockSpec((1, D), lambda i: (0, 0)),
    ],
    out_specs=pl.BlockSpec((_BLK, D), lambda i: (i, 0)),
    out_shape=jax.ShapeDtypeStruct((N, D), jnp.float32),
)

_tc_out = pl.pallas_call(
    _tc_out_body,
    grid=(N // _BLK,),
    in_specs=[
        pl.BlockSpec((_BLK, D), lambda i: (i, 0)),
        pl.BlockSpec((_BLK, D), lambda i: (i, 0)),
        pl.BlockSpec((D, D), lambda i: (0, 0)),
    ],
    out_specs=pl.BlockSpec((_BLK, D), lambda i: (i, 0)),
    out_shape=jax.ShapeDtypeStruct((N, D), jnp.float32),
)


def kernel(x, edge, neighbors, W_global, W_local, W_self, bias):
    nbr = jnp.concatenate(
        [neighbors, jnp.zeros((NPAD - N) * DEG, dtype=jnp.int32)]
    ).reshape(NW, PHASES, SPP, ROWS)
    z_self = _tc_self(x, W_global, W_self, bias.reshape(1, D))
    ns = _sc_neighbor_sum(x, nbr)
    return _tc_out(z_self, ns, W_local)


# speedup vs baseline: 6.6545x; 1.0446x over previous
import functools

import jax
import jax.numpy as jnp
from jax import lax
from jax.experimental import pallas as pl
from jax.experimental.pallas import tpu as pltpu
from jax.experimental.pallas import tpu_sc as plsc

N = 10000
DEG = 32
D = 128
LANES = 16
VECS = D // LANES

NW = 32
CPW = 320
NPAD = NW * CPW
ROWS = 64
NODES_PER_STEP = ROWS // DEG
STEPS = (CPW * DEG) // ROWS
PHASES = 4
SPP = STEPS // PHASES
HPW = CPW // PHASES
XSH = 10112


_sc_mesh = plsc.VectorSubcoreMesh(
    core_axis_name="c", subcore_axis_name="s", num_cores=2, num_subcores=16
)


@functools.partial(
    pl.kernel,
    out_type=jax.ShapeDtypeStruct((NPAD, D), jnp.float32),
    mesh=_sc_mesh,
    scratch_types=[
        pltpu.VMEM((STEPS, ROWS), jnp.int32),
        pltpu.VMEM((HPW, D), jnp.float32),
        pltpu.VMEM_SHARED((XSH, D), jnp.float32),
        pltpu.VMEM((ROWS, D), jnp.float32),
        pltpu.VMEM((ROWS, D), jnp.float32),
        pltpu.SemaphoreType.DMA,
        pltpu.SemaphoreType.DMA,
        pltpu.SemaphoreType.DMA,
    ],
)
def _sc_neighbor_sum(x_hbm, nbr_hbm, out_hbm,
                     nbr_v, res_v, x_sh, buf0, buf1, gsem0, gsem1, wsem):
    sid = lax.axis_index("s")
    wid = sid * 2 + lax.axis_index("c")
    @pl.when(sid < 15)
    def _():
        pltpu.sync_copy(x_hbm.at[pl.ds(sid * 632, 632)],
                        x_sh.at[pl.ds(sid * 632, 632)])

    @pl.when(sid == 15)
    def _():
        pltpu.sync_copy(x_hbm.at[pl.ds(15 * 632, N - 15 * 632)],
                        x_sh.at[pl.ds(15 * 632, N - 15 * 632)])

    plsc.subcore_barrier()

    def g_start(j, buf, sem):
        pltpu.async_copy(x_sh.at[nbr_v.at[j]], buf, sem)

    def g_wait(j, buf, sem):
        pltpu.make_async_copy(x_sh.at[nbr_v.at[j]], buf, sem).wait()

    def accumulate(j, buf):
        for n in range(NODES_PER_STEP):
            def grp(g, carry):
                base = n * DEG + g * 8
                new = []
                for c in range(VECS):
                    sl = pl.ds(c * LANES, LANES)
                    v = [buf[base + k, sl] for k in range(8)]
                    gs = ((v[0] + v[1]) + (v[2] + v[3])) + \
                         ((v[4] + v[5]) + (v[6] + v[7]))
                    new.append(carry[c] + gs)
                return tuple(new)

            tot = lax.fori_loop(
                0, DEG // 8, grp,
                tuple(jnp.zeros((LANES,), jnp.float32) for _ in range(VECS)))
            for c in range(VECS):
                res_v[NODES_PER_STEP * j + n, pl.ds(c * LANES, LANES)] = tot[c]

    pltpu.sync_copy(nbr_hbm.at[wid], nbr_v)

    def run_phase(ph):
        j0 = ph * SPP
        g_start(j0, buf0, gsem0)
        if ph > 0:
            pltpu.make_async_copy(
                res_v, out_hbm.at[pl.ds(wid * CPW, HPW)], wsem).wait()

        def pair(it, carry):
            a = j0 + 2 * it
            g_start(a + 1, buf1, gsem1)
            g_wait(a, buf0, gsem0)
            accumulate(2 * it, buf0)
            g_start(a + 2, buf0, gsem0)
            g_wait(a + 1, buf1, gsem1)
            accumulate(2 * it + 1, buf1)
            return carry

        lax.fori_loop(0, SPP // 2 - 1, pair, 0)

        g_start(j0 + SPP - 1, buf1, gsem1)
        g_wait(j0 + SPP - 2, buf0, gsem0)
        accumulate(SPP - 2, buf0)
        g_wait(j0 + SPP - 1, buf1, gsem1)
        accumulate(SPP - 1, buf1)

        pltpu.async_copy(res_v,
                         out_hbm.at[pl.ds(wid * CPW + ph * HPW, HPW)], wsem)

    for _ph in range(PHASES):
        run_phase(_ph)
    pltpu.make_async_copy(
        res_v, out_hbm.at[pl.ds(wid * CPW, HPW)], wsem).wait()


def _tc_self_body(x_ref, wg_ref, ws_ref, b_ref, o_ref):
    w_sum = wg_ref[...] + ws_ref[...]
    o_ref[...] = lax.dot_general(x_ref[...], w_sum, (((1,), (1,)), ((), ())),
                                 preferred_element_type=jnp.float32) + b_ref[...]


def _tc_out_body(z_ref, ns_ref, wl_ref, o_ref):
    z = z_ref[...] + lax.dot_general(ns_ref[...] * (1.0 / DEG), wl_ref[...],
                                     (((1,), (1,)), ((), ())),
                                     preferred_element_type=jnp.float32)
    o_ref[...] = jnp.where(z > 0, z, jnp.exp(z) - 1.0)


_BLK = 2000

_tc_self = pl.pallas_call(
    _tc_self_body,
    grid=(N // _BLK,),
    in_specs=[
        pl.BlockSpec((_BLK, D), lambda i: (i, 0)),
        pl.BlockSpec((D, D), lambda i: (0, 0)),
        pl.BlockSpec((D, D), lambda i: (0, 0)),
        pl.BlockSpec((1, D), lambda i: (0, 0)),
    ],
    out_specs=pl.BlockSpec((_BLK, D), lambda i: (i, 0)),
    out_shape=jax.ShapeDtypeStruct((N, D), jnp.float32),
)

_tc_out = pl.pallas_call(
    _tc_out_body,
    grid=(N // _BLK,),
    in_specs=[
        pl.BlockSpec((_BLK, D), lambda i: (i, 0)),
        pl.BlockSpec((_BLK, D), lambda i: (i, 0)),
        pl.BlockSpec((D, D), lambda i: (0, 0)),
    ],
    out_specs=pl.BlockSpec((_BLK, D), lambda i: (i, 0)),
    out_shape=jax.ShapeDtypeStruct((N, D), jnp.float32),
)


def kernel(x, edge, neighbors, W_global, W_local, W_self, bias):
    nbr = jnp.concatenate(
        [neighbors, jnp.zeros((NPAD - N) * DEG, dtype=jnp.int32)]
    ).reshape(NW, STEPS, ROWS)
    z_self = _tc_self(x, W_global, W_self, bias.reshape(1, D))
    ns = _sc_neighbor_sum(x, nbr)
    return _tc_out(z_self, ns, W_local)


# speedup vs baseline: 6.6866x; 1.0048x over previous
import functools

import jax
import jax.numpy as jnp
from jax import lax
from jax.experimental import pallas as pl
from jax.experimental.pallas import tpu as pltpu
from jax.experimental.pallas import tpu_sc as plsc

N = 10000
DEG = 32
D = 128
LANES = 16
VECS = D // LANES

NW = 32
CPW = 320
NPAD = NW * CPW
ROWS = 64
NODES_PER_STEP = ROWS // DEG
STEPS = (CPW * DEG) // ROWS
PHASES = 4
SPP = STEPS // PHASES
HPW = CPW // PHASES
XSH = 10112


_sc_mesh = plsc.VectorSubcoreMesh(
    core_axis_name="c", subcore_axis_name="s", num_cores=2, num_subcores=16
)


@functools.partial(
    pl.kernel,
    out_type=jax.ShapeDtypeStruct((NPAD, D), jnp.float32),
    mesh=_sc_mesh,
    scratch_types=[
        pltpu.VMEM((STEPS, ROWS), jnp.int32),
        pltpu.VMEM((HPW, D), jnp.float32),
        pltpu.VMEM_SHARED((XSH, D), jnp.float32),
        pltpu.VMEM((ROWS, D), jnp.float32),
        pltpu.VMEM((ROWS, D), jnp.float32),
        pltpu.SemaphoreType.DMA,
        pltpu.SemaphoreType.DMA,
        pltpu.SemaphoreType.DMA,
    ],
)
def _sc_neighbor_sum(x_hbm, nbr_hbm, out_hbm,
                     nbr_v, res_v, x_sh, buf0, buf1, gsem0, gsem1, wsem):
    sid = lax.axis_index("s")
    wid = sid * 2 + lax.axis_index("c")
    @pl.when(sid < 15)
    def _():
        pltpu.sync_copy(x_hbm.at[pl.ds(sid * 632, 632)],
                        x_sh.at[pl.ds(sid * 632, 632)])

    @pl.when(sid == 15)
    def _():
        pltpu.sync_copy(x_hbm.at[pl.ds(15 * 632, N - 15 * 632)],
                        x_sh.at[pl.ds(15 * 632, N - 15 * 632)])

    plsc.subcore_barrier()

    def g_start(j, buf, sem):
        pltpu.async_copy(x_sh.at[nbr_v.at[j]], buf, sem)

    def g_wait(j, buf, sem):
        pltpu.make_async_copy(x_sh.at[nbr_v.at[j]], buf, sem).wait()

    def accumulate(j, buf):
        for n in range(NODES_PER_STEP):
            def grp(g, carry):
                base = n * DEG + g * 8
                new = []
                for c in range(VECS):
                    sl = pl.ds(c * LANES, LANES)
                    v = [buf[base + k, sl] for k in range(8)]
                    gs = ((v[0] + v[1]) + (v[2] + v[3])) + \
                         ((v[4] + v[5]) + (v[6] + v[7]))
                    new.append(carry[c] + gs)
                return tuple(new)

            tot = lax.fori_loop(
                0, DEG // 8, grp,
                tuple(jnp.zeros((LANES,), jnp.float32) for _ in range(VECS)))
            for c in range(VECS):
                res_v[NODES_PER_STEP * j + n, pl.ds(c * LANES, LANES)] = tot[c]

    pltpu.sync_copy(nbr_hbm.at[wid], nbr_v)

    def run_phase(ph):
        j0 = ph * SPP
        g_start(j0, buf0, gsem0)
        if ph > 0:
            pltpu.make_async_copy(
                res_v, out_hbm.at[pl.ds(wid * CPW, HPW)], wsem).wait()

        def pair(it, carry):
            a = j0 + 2 * it
            g_start(a + 1, buf1, gsem1)
            g_wait(a, buf0, gsem0)
            accumulate(2 * it, buf0)
            g_start(a + 2, buf0, gsem0)
            g_wait(a + 1, buf1, gsem1)
            accumulate(2 * it + 1, buf1)
            return carry

        lax.fori_loop(0, SPP // 2 - 1, pair, 0)

        g_start(j0 + SPP - 1, buf1, gsem1)
        g_wait(j0 + SPP - 2, buf0, gsem0)
        accumulate(SPP - 2, buf0)
        g_wait(j0 + SPP - 1, buf1, gsem1)
        accumulate(SPP - 1, buf1)

        pltpu.async_copy(res_v,
                         out_hbm.at[pl.ds(wid * CPW + ph * HPW, HPW)], wsem)

    for _ph in range(PHASES):
        run_phase(_ph)
    pltpu.make_async_copy(
        res_v, out_hbm.at[pl.ds(wid * CPW, HPW)], wsem).wait()


def _tc_body(x_ref, ns_ref, wg_ref, wl_ref, ws_ref, b_ref, o_ref):
    w_sum = wg_ref[...] + ws_ref[...]
    z = lax.dot_general(x_ref[...], w_sum, (((1,), (1,)), ((), ())),
                        preferred_element_type=jnp.float32)
    z += lax.dot_general(ns_ref[...] * (1.0 / DEG), wl_ref[...],
                         (((1,), (1,)), ((), ())),
                         preferred_element_type=jnp.float32)
    z += b_ref[...]
    o_ref[...] = jnp.where(z > 0, z, jnp.exp(z) - 1.0)


_BLK = 2000

_tc_fuse = pl.pallas_call(
    _tc_body,
    grid=(N // _BLK,),
    in_specs=[
        pl.BlockSpec((_BLK, D), lambda i: (i, 0)),
        pl.BlockSpec((_BLK, D), lambda i: (i, 0)),
        pl.BlockSpec((D, D), lambda i: (0, 0)),
        pl.BlockSpec((D, D), lambda i: (0, 0)),
        pl.BlockSpec((D, D), lambda i: (0, 0)),
        pl.BlockSpec((1, D), lambda i: (0, 0)),
    ],
    out_specs=pl.BlockSpec((_BLK, D), lambda i: (i, 0)),
    out_shape=jax.ShapeDtypeStruct((N, D), jnp.float32),
)


def kernel(x, edge, neighbors, W_global, W_local, W_self, bias):
    nbr = jnp.concatenate(
        [neighbors, jnp.zeros((NPAD - N) * DEG, dtype=jnp.int32)]
    ).reshape(NW, STEPS, ROWS)
    ns = _sc_neighbor_sum(x, nbr)
    return _tc_fuse(x, ns, W_global, W_local, W_self, bias.reshape(1, D))
